# Initial kernel scaffold; baseline (speedup 1.0000x reference)
import jax
import jax.numpy as jnp
from jax.experimental import pallas as pl


def kernel(query, query_pos, feat0, feat1, feat2, feat3, lidar2img, reference_points, img_size, so_w, so_b, aw_w, aw_b, vp_w, vp_b, out_w, out_b):
    raise NotImplementedError("write your pallas kernel here")


# speedup vs baseline: 26.1315x; 26.1315x over previous
import jax
import jax.numpy as jnp
import numpy as np
from jax import lax
from jax.experimental import pallas as pl
from jax.experimental.pallas import tpu as pltpu
from jax.experimental.pallas import tpu_sc as plsc

B = 1
NQ = 2500
NQP = 2560
ED = 256
NH = 8
NL = 4
NP = 4
NC = 6
HD = ED // NH
SS = np.array([[64, 112], [32, 56], [16, 28], [8, 14]], dtype=np.int64)
LSTART = np.array([0, 7168, 8960, 9408], dtype=np.int64)
NV = 9520
NVP = 9600
NK = NL * NP * 4

ITEMS = NC * NH * NQP
NSUB = 32
IT_PER = ITEMS // NSUB
IB = 16
NBLK = IT_PER // IB

QB1 = 256
QB2 = 256
NVB = 640
QBD = 256

_f32 = jnp.float32
_i32 = jnp.int32


def _a1_body(qr, pr, wxr, bxr, wyr, byr, war, bar, oxr, oyr, oar):
    q = qr[...] + pr[...]
    dn = (((1,), (1,)), ((), ()))
    offx = lax.dot_general(q, wxr[0], dn, preferred_element_type=_f32) + bxr[0]
    offy = lax.dot_general(q, wyr[0], dn, preferred_element_type=_f32) + byr[0]
    a = lax.dot_general(q, war[0], dn, preferred_element_type=_f32) + bar[0]
    m = jnp.max(a, axis=1, keepdims=True)
    e = jnp.exp(a - m)
    s = jnp.sum(e, axis=1, keepdims=True)
    oxr[0] = offx
    oyr[0] = offy
    oar[0] = e * (4.0 / s)


def _run_a1(qpad, ppad, sowx, sobx, sowy, soby, aww, awb):
    grid = (NH, NQP // QB1)
    return pl.pallas_call(
        _a1_body,
        grid=grid,
        in_specs=[
            pl.BlockSpec((QB1, ED), lambda h, qi: (qi, 0)),
            pl.BlockSpec((QB1, ED), lambda h, qi: (qi, 0)),
            pl.BlockSpec((1, NK, ED), lambda h, qi: (h, 0, 0)),
            pl.BlockSpec((1, 1, NK), lambda h, qi: (h, 0, 0)),
            pl.BlockSpec((1, NK, ED), lambda h, qi: (h, 0, 0)),
            pl.BlockSpec((1, 1, NK), lambda h, qi: (h, 0, 0)),
            pl.BlockSpec((1, NK, ED), lambda h, qi: (h, 0, 0)),
            pl.BlockSpec((1, 1, NK), lambda h, qi: (h, 0, 0)),
        ],
        out_specs=[
            pl.BlockSpec((1, QB1, NK), lambda h, qi: (h, qi, 0)),
            pl.BlockSpec((1, QB1, NK), lambda h, qi: (h, qi, 0)),
            pl.BlockSpec((1, QB1, NK), lambda h, qi: (h, qi, 0)),
        ],
        out_shape=[jax.ShapeDtypeStruct((NH, NQP, NK), _f32)] * 3,
    )(qpad, ppad, sowx, sobx, sowy, soby, aww, awb)


def _a2_body(oxr, oyr, awr, rpr, l2ir, imgr, wlr, hlr, dxr, dyr, str_,
             idxr, wr, mr):
    c = pl.program_id(0)
    h = pl.program_id(2)
    rp = rpr[...]
    cam = []
    for i in range(3):
        acc = rp[:, 0:1] * l2ir[c, i, 0]
        for j in range(1, 4):
            acc = acc + rp[:, j:j + 1] * l2ir[c, i, j]
        cam.append(acc)
    depth = cam[2]
    denom = depth + 1e-5
    img_h = imgr[0].astype(_f32)
    img_w = imgr[1].astype(_f32)
    un = cam[0] / denom / img_w
    vn = cam[1] / denom / img_h
    maskf = ((depth > 1e-5) & (un > 0.0) & (un < 1.0)
             & (vn > 0.0) & (vn < 1.0)).astype(_f32)

    wl = wlr[...]
    hl = hlr[...]
    dxl = dxr[...]
    dyl = dyr[...]
    stl = str_[...]

    x = un * wl + oxr[0] - 0.5
    y = vn * hl + oyr[0] - 0.5
    x0 = jnp.floor(x)
    y0 = jnp.floor(y)
    fx = x - x0
    fy = y - y0
    xc = x0 + dxl
    yc = y0 + dyl
    wx = jnp.where(dxl > 0.5, fx, 1.0 - fx)
    wy = jnp.where(dyl > 0.5, fy, 1.0 - fy)
    inb = ((xc >= 0.0) & (xc <= wl - 1.0)
           & (yc >= 0.0) & (yc <= hl - 1.0)).astype(_f32)
    wtot = awr[0] * wx * wy * inb * maskf
    xci = jnp.clip(xc, 0.0, wl - 1.0)
    yci = jnp.clip(yc, 0.0, hl - 1.0)
    row_l = stl + yci * wl + xci
    gidx = row_l.astype(_i32) * NH + (c * NVP * NH + h)
    idxr[0, 0] = gidx
    wr[0, 0] = wtot
    mr[0] = maskf


def _run_a2(offx, offy, aw, rp_pad, l2i, img, lane_consts):
    wl, hl, dxl, dyl, stl = lane_consts
    grid = (NC, NQP // QB2, NH)
    lane_spec = pl.BlockSpec((1, NK), lambda c, qi, h: (0, 0))
    return pl.pallas_call(
        _a2_body,
        grid=grid,
        in_specs=[
            pl.BlockSpec((1, QB2, NK), lambda c, qi, h: (h, qi, 0)),
            pl.BlockSpec((1, QB2, NK), lambda c, qi, h: (h, qi, 0)),
            pl.BlockSpec((1, QB2, NK), lambda c, qi, h: (h, qi, 0)),
            pl.BlockSpec((QB2, 4), lambda c, qi, h: (qi, 0)),
            pl.BlockSpec(memory_space=pltpu.SMEM),
            pl.BlockSpec(memory_space=pltpu.SMEM),
            lane_spec, lane_spec, lane_spec, lane_spec, lane_spec,
        ],
        out_specs=[
            pl.BlockSpec((1, 1, QB2, NK), lambda c, qi, h: (c, h, qi, 0)),
            pl.BlockSpec((1, 1, QB2, NK), lambda c, qi, h: (c, h, qi, 0)),
            pl.BlockSpec((1, QB2, 1), lambda c, qi, h: (c, qi, 0)),
        ],
        out_shape=[
            jax.ShapeDtypeStruct((NC, NH, NQP, NK), _i32),
            jax.ShapeDtypeStruct((NC, NH, NQP, NK), _f32),
            jax.ShapeDtypeStruct((NC, NQP, 1), _f32),
        ],
    )(offx, offy, aw, rp_pad, l2i, img, wl, hl, dxl, dyl, stl)


def _b_body(fr, vwr, vbr, outr):
    dn = (((0,), (1,)), ((), ()))
    outr[0] = lax.dot_general(fr[0], vwr[...], dn,
                              preferred_element_type=_f32) + vbr[...]


def _run_b(ftpad, vp_w, vp_b2):
    grid = (NC, NVP // NVB)
    return pl.pallas_call(
        _b_body,
        grid=grid,
        in_specs=[
            pl.BlockSpec((1, ED, NVB), lambda c, vi: (c, 0, vi)),
            pl.BlockSpec((ED, ED), lambda c, vi: (0, 0)),
            pl.BlockSpec((1, ED), lambda c, vi: (0, 0)),
        ],
        out_specs=pl.BlockSpec((1, NVB, ED), lambda c, vi: (c, vi, 0)),
        out_shape=jax.ShapeDtypeStruct((NC, NVP, ED), _f32),
    )(ftpad, vp_w, vp_b2)


def _sc_body(idx_hbm, w_hbm, table_hbm, out_hbm, idxv, wv, rowsv, outv, gsem):
    cid = lax.axis_index("c")
    sid = lax.axis_index("s")
    wid = sid * 2 + cid
    base = wid * IT_PER

    @pl.loop(0, NBLK)
    def _blk(b):
        ib = base + b * IB
        pltpu.sync_copy(idx_hbm.at[pl.ds(ib * NK, IB * NK)], idxv)
        pltpu.sync_copy(w_hbm.at[pl.ds(ib * NK, IB * NK)], wv)
        handles = []
        for g in range(IB * NK // 128):
            handles.append(pltpu.async_copy(
                table_hbm.at[idxv.at[pl.ds(g * 128, 128)]],
                rowsv.at[pl.ds(g * 128, 128)], gsem))
        for hnd in handles:
            hnd.wait()

        @pl.loop(0, IB)
        def _item(i):
            def kstep(k, acc):
                a0, a1 = acc
                flat = i * NK + k
                wk = plsc.load_gather(wv, [jnp.full((16,), flat, _i32)])
                r0 = rowsv[flat, pl.ds(0, 16)]
                r1 = rowsv[flat, pl.ds(16, 16)]
                return (a0 + wk * r0, a1 + wk * r1)

            a0, a1 = lax.fori_loop(
                0, NK, kstep,
                (jnp.zeros((16,), _f32), jnp.zeros((16,), _f32)))
            outv[i, pl.ds(0, 16)] = a0
            outv[i, pl.ds(16, 16)] = a1

        pltpu.sync_copy(outv, out_hbm.at[pl.ds(ib, IB)])


def _run_sc(idx_flat, w_flat, table):
    mesh = plsc.VectorSubcoreMesh(core_axis_name="c", subcore_axis_name="s")
    f = pl.kernel(
        _sc_body,
        out_type=jax.ShapeDtypeStruct((ITEMS, HD), _f32),
        mesh=mesh,
        scratch_types=[
            pltpu.VMEM((IB * NK,), _i32),
            pltpu.VMEM((IB * NK,), _f32),
            pltpu.VMEM((IB * NK, HD), _f32),
            pltpu.VMEM((IB, HD), _f32),
            pltpu.SemaphoreType.DMA,
        ],
        compiler_params=pltpu.CompilerParams(
            needs_layout_passes=False, use_tc_tiling_on_sc=False),
    )
    return f(idx_flat, w_flat, table)


def _d_body(prer, mr, owr, obr, outr):
    h = pl.program_id(1)
    fused = jnp.sum(prer[...], axis=0)[0]
    m = mr[...]
    valid = jnp.sum(m, axis=0)
    scale = 1.0 / jnp.maximum(valid, 1.0)
    ind = (valid > 0.0).astype(_f32)
    part = jnp.dot(fused * scale, owr[...], preferred_element_type=_f32)

    @pl.when(h == 0)
    def _():
        outr[...] = part + ind * obr[...]

    @pl.when(h != 0)
    def _():
        outr[...] = outr[...] + part


def _run_d(pre, maskq, out_w_t, out_b2):
    grid = (NQP // QBD, NH)
    return pl.pallas_call(
        _d_body,
        grid=grid,
        in_specs=[
            pl.BlockSpec((NC, 1, QBD, HD), lambda qi, h: (0, h, qi, 0)),
            pl.BlockSpec((NC, QBD, 1), lambda qi, h: (0, qi, 0)),
            pl.BlockSpec((HD, ED), lambda qi, h: (h, 0)),
            pl.BlockSpec((1, ED), lambda qi, h: (0, 0)),
        ],
        out_specs=pl.BlockSpec((QBD, ED), lambda qi, h: (qi, 0)),
        out_shape=jax.ShapeDtypeStruct((NQP, ED), _f32),
    )(pre, maskq, out_w_t, out_b2)


def kernel(query, query_pos, feat0, feat1, feat2, feat3, lidar2img,
           reference_points, img_size, so_w, so_b, aw_w, aw_b, vp_w, vp_b,
           out_w, out_b):
    f32 = _f32
    qpad = jnp.concatenate(
        [query[0], jnp.zeros((NQP - NQ, ED), f32)], axis=0)
    ppad = jnp.concatenate(
        [query_pos[0], jnp.zeros((NQP - NQ, ED), f32)], axis=0)
    def _round_bf16(x):
        u = jax.lax.bitcast_convert_type(x, jnp.uint32)
        u = (u + jnp.uint32(0x7FFF) + ((u >> 16) & jnp.uint32(1)))
        u = u & jnp.uint32(0xFFFF0000)
        return jax.lax.bitcast_convert_type(u, f32)

    rp1 = jnp.concatenate(
        [reference_points, jnp.ones((NQ, 1), f32)], axis=1)
    rp1 = _round_bf16(rp1)
    rp_pad = jnp.concatenate([rp1, jnp.zeros((NQP - NQ, 4), f32)], axis=0)

    kk = np.arange(NK)
    ll = kk // 16
    pp = (kk // 4) % 4
    cc = kk % 4
    dx = (cc % 2).astype(np.float32)
    dy = (cc // 2).astype(np.float32)
    hh = np.arange(NH)[:, None]
    rx = (((hh * NL + ll[None]) * NP + pp[None]) * 2 + 0).astype(np.int32)
    ra = ((hh * NL + ll[None]) * NP + pp[None]).astype(np.int32)
    sowx = so_w[rx]
    sowy = so_w[rx + 1]
    sobx = so_b[rx][:, None, :]
    soby = so_b[rx + 1][:, None, :]
    aww = aw_w[ra]
    awb = aw_b[ra][:, None, :]

    wl = jnp.asarray(SS[ll, 1], f32)[None, :]
    hl = jnp.asarray(SS[ll, 0], f32)[None, :]
    dxl = jnp.asarray(dx)[None, :]
    dyl = jnp.asarray(dy)[None, :]
    stl = jnp.asarray(LSTART[ll], f32)[None, :]

    feats = [feat0, feat1, feat2, feat3]
    ft = jnp.concatenate(
        [feats[l][0].reshape(NC, ED, -1) for l in range(NL)], axis=2)
    ftpad = jnp.concatenate(
        [ft, jnp.zeros((NC, ED, NVP - NV), f32)], axis=2)

    l2i = _round_bf16(lidar2img[0])
    vp_b2 = vp_b[None, :]
    out_b2 = out_b[None, :]
    out_w_t = out_w.T

    offx, offy, aw = _run_a1(qpad, ppad, sowx, sobx, sowy, soby, aww, awb)
    idx, w, maskq = _run_a2(offx, offy, aw, rp_pad, l2i, img_size,
                            (wl, hl, dxl, dyl, stl))
    table = _run_b(ftpad, vp_w, vp_b2)
    table2 = table.reshape(NC * NVP * NH, HD)
    pre = _run_sc(idx.reshape(-1), w.reshape(-1), table2)
    pre4 = pre.reshape(NC, NH, NQP, HD)
    res = _run_d(pre4, maskq, out_w_t, out_b2)
    return res[:NQ][None]


# speedup vs baseline: 27.5915x; 1.0559x over previous
import jax
import jax.numpy as jnp
import numpy as np
from jax import lax
from jax.experimental import pallas as pl
from jax.experimental.pallas import tpu as pltpu
from jax.experimental.pallas import tpu_sc as plsc

B = 1
NQ = 2500
NQP = 2560
ED = 256
NH = 8
NL = 4
NP = 4
NC = 6
HD = ED // NH
SS = np.array([[64, 112], [32, 56], [16, 28], [8, 14]], dtype=np.int64)
LSTART = np.array([0, 7168, 8960, 9408], dtype=np.int64)
NV = 9520
NVP = 9600
NK = NL * NP * 4

ITEMS = NC * NH * NQP
NSUB = 32
IT_PER = ITEMS // NSUB
IB = 16
NBLK = IT_PER // IB

QB1 = 256
QB2 = 256
NVB = 640
QBD = 256

_f32 = jnp.float32
_i32 = jnp.int32


def _a1_body(qr, pr, wxr, bxr, wyr, byr, war, bar, oxr, oyr, oar):
    q = qr[...] + pr[...]
    dn = (((1,), (1,)), ((), ()))
    offx = lax.dot_general(q, wxr[0], dn, preferred_element_type=_f32) + bxr[0]
    offy = lax.dot_general(q, wyr[0], dn, preferred_element_type=_f32) + byr[0]
    a = lax.dot_general(q, war[0], dn, preferred_element_type=_f32) + bar[0]
    m = jnp.max(a, axis=1, keepdims=True)
    e = jnp.exp(a - m)
    s = jnp.sum(e, axis=1, keepdims=True)
    oxr[0] = offx
    oyr[0] = offy
    oar[0] = e * (4.0 / s)


def _run_a1(qpad, ppad, sowx, sobx, sowy, soby, aww, awb):
    grid = (NH, NQP // QB1)
    return pl.pallas_call(
        _a1_body,
        grid=grid,
        in_specs=[
            pl.BlockSpec((QB1, ED), lambda h, qi: (qi, 0)),
            pl.BlockSpec((QB1, ED), lambda h, qi: (qi, 0)),
            pl.BlockSpec((1, NK, ED), lambda h, qi: (h, 0, 0)),
            pl.BlockSpec((1, 1, NK), lambda h, qi: (h, 0, 0)),
            pl.BlockSpec((1, NK, ED), lambda h, qi: (h, 0, 0)),
            pl.BlockSpec((1, 1, NK), lambda h, qi: (h, 0, 0)),
            pl.BlockSpec((1, NK, ED), lambda h, qi: (h, 0, 0)),
            pl.BlockSpec((1, 1, NK), lambda h, qi: (h, 0, 0)),
        ],
        out_specs=[
            pl.BlockSpec((1, QB1, NK), lambda h, qi: (h, qi, 0)),
            pl.BlockSpec((1, QB1, NK), lambda h, qi: (h, qi, 0)),
            pl.BlockSpec((1, QB1, NK), lambda h, qi: (h, qi, 0)),
        ],
        out_shape=[jax.ShapeDtypeStruct((NH, NQP, NK), _f32)] * 3,
    )(qpad, ppad, sowx, sobx, sowy, soby, aww, awb)


def _a2_body(oxr, oyr, awr, rpr, l2ir, imgr, wlr, hlr, dxr, dyr, str_,
             idxr, wr, mr):
    c = pl.program_id(0)
    h = pl.program_id(2)
    rp = rpr[...]
    cam = []
    for i in range(3):
        acc = rp[:, 0:1] * l2ir[c, i, 0]
        for j in range(1, 4):
            acc = acc + rp[:, j:j + 1] * l2ir[c, i, j]
        cam.append(acc)
    depth = cam[2]
    denom = depth + 1e-5
    img_h = imgr[0].astype(_f32)
    img_w = imgr[1].astype(_f32)
    un = cam[0] / denom / img_w
    vn = cam[1] / denom / img_h
    maskf = ((depth > 1e-5) & (un > 0.0) & (un < 1.0)
             & (vn > 0.0) & (vn < 1.0)).astype(_f32)

    wl = wlr[...]
    hl = hlr[...]
    dxl = dxr[...]
    dyl = dyr[...]
    stl = str_[...]

    x = un * wl + oxr[0] - 0.5
    y = vn * hl + oyr[0] - 0.5
    x0 = jnp.floor(x)
    y0 = jnp.floor(y)
    fx = x - x0
    fy = y - y0
    xc = x0 + dxl
    yc = y0 + dyl
    wx = jnp.where(dxl > 0.5, fx, 1.0 - fx)
    wy = jnp.where(dyl > 0.5, fy, 1.0 - fy)
    inb = ((xc >= 0.0) & (xc <= wl - 1.0)
           & (yc >= 0.0) & (yc <= hl - 1.0)).astype(_f32)
    wtot = awr[0] * wx * wy * inb * maskf
    xci = jnp.clip(xc, 0.0, wl - 1.0)
    yci = jnp.clip(yc, 0.0, hl - 1.0)
    row_l = stl + yci * wl + xci
    gidx = row_l.astype(_i32) * NH + (c * NVP * NH + h)
    idxr[0, 0] = gidx
    wr[0, 0] = wtot
    mr[0] = maskf


def _run_a2(offx, offy, aw, rp_pad, l2i, img, lane_consts):
    wl, hl, dxl, dyl, stl = lane_consts
    grid = (NC, NQP // QB2, NH)
    lane_spec = pl.BlockSpec((1, NK), lambda c, qi, h: (0, 0))
    return pl.pallas_call(
        _a2_body,
        grid=grid,
        in_specs=[
            pl.BlockSpec((1, QB2, NK), lambda c, qi, h: (h, qi, 0)),
            pl.BlockSpec((1, QB2, NK), lambda c, qi, h: (h, qi, 0)),
            pl.BlockSpec((1, QB2, NK), lambda c, qi, h: (h, qi, 0)),
            pl.BlockSpec((QB2, 4), lambda c, qi, h: (qi, 0)),
            pl.BlockSpec(memory_space=pltpu.SMEM),
            pl.BlockSpec(memory_space=pltpu.SMEM),
            lane_spec, lane_spec, lane_spec, lane_spec, lane_spec,
        ],
        out_specs=[
            pl.BlockSpec((1, 1, QB2, NK), lambda c, qi, h: (c, h, qi, 0)),
            pl.BlockSpec((1, 1, QB2, NK), lambda c, qi, h: (c, h, qi, 0)),
            pl.BlockSpec((1, QB2, 1), lambda c, qi, h: (c, qi, 0)),
        ],
        out_shape=[
            jax.ShapeDtypeStruct((NC, NH, NQP, NK), _i32),
            jax.ShapeDtypeStruct((NC, NH, NQP, NK), _f32),
            jax.ShapeDtypeStruct((NC, NQP, 1), _f32),
        ],
    )(offx, offy, aw, rp_pad, l2i, img, wl, hl, dxl, dyl, stl)


def _b_body(fr, vwr, vbr, outr):
    dn = (((0,), (1,)), ((), ()))
    outr[0] = lax.dot_general(fr[0], vwr[...], dn,
                              preferred_element_type=_f32) + vbr[...]


def _run_b(ftpad, vp_w, vp_b2):
    grid = (NC, NVP // NVB)
    return pl.pallas_call(
        _b_body,
        grid=grid,
        in_specs=[
            pl.BlockSpec((1, ED, NVB), lambda c, vi: (c, 0, vi)),
            pl.BlockSpec((ED, ED), lambda c, vi: (0, 0)),
            pl.BlockSpec((1, ED), lambda c, vi: (0, 0)),
        ],
        out_specs=pl.BlockSpec((1, NVB, ED), lambda c, vi: (c, vi, 0)),
        out_shape=jax.ShapeDtypeStruct((NC, NVP, ED), _f32),
    )(ftpad, vp_w, vp_b2)


def _sc_body(idx_hbm, w_hbm, table_hbm, out_hbm,
             idxv0, wv0, rowsv0, idxv1, wv1, rowsv1, outv, sem0, sem1):
    cid = lax.axis_index("c")
    sid = lax.axis_index("s")
    wid = sid * 2 + cid
    base = wid * IT_PER
    bufs = ((idxv0, wv0, rowsv0, sem0), (idxv1, wv1, rowsv1, sem1))

    def fire(b, s):
        idxv, wv, rowsv, sem = bufs[s]
        ib = base + b * IB
        pltpu.sync_copy(idx_hbm.at[pl.ds(ib * NK, IB * NK)], idxv)
        pltpu.sync_copy(w_hbm.at[pl.ds(ib * NK, IB * NK)], wv)
        for g in range(IB * NK // 128):
            pltpu.async_copy(
                table_hbm.at[idxv.at[pl.ds(g * 128, 128)]],
                rowsv.at[pl.ds(g * 128, 128)], sem)

    def consume(b, s):
        idxv, wv, rowsv, sem = bufs[s]
        pltpu.make_async_copy(
            table_hbm.at[pl.ds(0, IB * NK)], rowsv, sem).wait()

        @pl.loop(0, IB)
        def _item(i):
            def kstep(k, acc):
                a0, a1 = acc
                flat = i * NK + k
                wk = plsc.load_gather(wv, [jnp.full((16,), flat, _i32)])
                r0 = rowsv[flat, pl.ds(0, 16)]
                r1 = rowsv[flat, pl.ds(16, 16)]
                return (a0 + wk * r0, a1 + wk * r1)

            a0, a1 = lax.fori_loop(
                0, NK, kstep,
                (jnp.zeros((16,), _f32), jnp.zeros((16,), _f32)))
            outv[i, pl.ds(0, 16)] = a0
            outv[i, pl.ds(16, 16)] = a1

        pltpu.sync_copy(outv, out_hbm.at[pl.ds(base + b * IB, IB)])

    fire(0, 0)
    fire(1, 1)

    @pl.loop(0, NBLK // 2)
    def _pair(bb):
        for s in range(2):
            b = bb * 2 + s
            consume(b, s)

            @pl.when(b + 2 < NBLK)
            def _():
                fire(b + 2, s)


def _run_sc(idx_flat, w_flat, table):
    mesh = plsc.VectorSubcoreMesh(core_axis_name="c", subcore_axis_name="s")
    f = pl.kernel(
        _sc_body,
        out_type=jax.ShapeDtypeStruct((ITEMS, HD), _f32),
        mesh=mesh,
        scratch_types=[
            pltpu.VMEM((IB * NK,), _i32),
            pltpu.VMEM((IB * NK,), _f32),
            pltpu.VMEM((IB * NK, HD), _f32),
            pltpu.VMEM((IB * NK,), _i32),
            pltpu.VMEM((IB * NK,), _f32),
            pltpu.VMEM((IB * NK, HD), _f32),
            pltpu.VMEM((IB, HD), _f32),
            pltpu.SemaphoreType.DMA,
            pltpu.SemaphoreType.DMA,
        ],
        compiler_params=pltpu.CompilerParams(
            needs_layout_passes=False, use_tc_tiling_on_sc=False),
    )
    return f(idx_flat, w_flat, table)


def _d_body(prer, mr, owr, obr, outr):
    h = pl.program_id(1)
    fused = jnp.sum(prer[...], axis=0)[0]
    m = mr[...]
    valid = jnp.sum(m, axis=0)
    scale = 1.0 / jnp.maximum(valid, 1.0)
    ind = (valid > 0.0).astype(_f32)
    part = jnp.dot(fused * scale, owr[...], preferred_element_type=_f32)

    @pl.when(h == 0)
    def _():
        outr[...] = part + ind * obr[...]

    @pl.when(h != 0)
    def _():
        outr[...] = outr[...] + part


def _run_d(pre, maskq, out_w_t, out_b2):
    grid = (NQP // QBD, NH)
    return pl.pallas_call(
        _d_body,
        grid=grid,
        in_specs=[
            pl.BlockSpec((NC, 1, QBD, HD), lambda qi, h: (0, h, qi, 0)),
            pl.BlockSpec((NC, QBD, 1), lambda qi, h: (0, qi, 0)),
            pl.BlockSpec((HD, ED), lambda qi, h: (h, 0)),
            pl.BlockSpec((1, ED), lambda qi, h: (0, 0)),
        ],
        out_specs=pl.BlockSpec((QBD, ED), lambda qi, h: (qi, 0)),
        out_shape=jax.ShapeDtypeStruct((NQP, ED), _f32),
    )(pre, maskq, out_w_t, out_b2)


def kernel(query, query_pos, feat0, feat1, feat2, feat3, lidar2img,
           reference_points, img_size, so_w, so_b, aw_w, aw_b, vp_w, vp_b,
           out_w, out_b):
    f32 = _f32
    qpad = jnp.concatenate(
        [query[0], jnp.zeros((NQP - NQ, ED), f32)], axis=0)
    ppad = jnp.concatenate(
        [query_pos[0], jnp.zeros((NQP - NQ, ED), f32)], axis=0)
    def _round_bf16(x):
        u = jax.lax.bitcast_convert_type(x, jnp.uint32)
        u = (u + jnp.uint32(0x7FFF) + ((u >> 16) & jnp.uint32(1)))
        u = u & jnp.uint32(0xFFFF0000)
        return jax.lax.bitcast_convert_type(u, f32)

    rp1 = jnp.concatenate(
        [reference_points, jnp.ones((NQ, 1), f32)], axis=1)
    rp1 = _round_bf16(rp1)
    rp_pad = jnp.concatenate([rp1, jnp.zeros((NQP - NQ, 4), f32)], axis=0)

    kk = np.arange(NK)
    ll = kk // 16
    pp = (kk // 4) % 4
    cc = kk % 4
    dx = (cc % 2).astype(np.float32)
    dy = (cc // 2).astype(np.float32)
    hh = np.arange(NH)[:, None]
    rx = (((hh * NL + ll[None]) * NP + pp[None]) * 2 + 0).astype(np.int32)
    ra = ((hh * NL + ll[None]) * NP + pp[None]).astype(np.int32)
    sowx = so_w[rx]
    sowy = so_w[rx + 1]
    sobx = so_b[rx][:, None, :]
    soby = so_b[rx + 1][:, None, :]
    aww = aw_w[ra]
    awb = aw_b[ra][:, None, :]

    wl = jnp.asarray(SS[ll, 1], f32)[None, :]
    hl = jnp.asarray(SS[ll, 0], f32)[None, :]
    dxl = jnp.asarray(dx)[None, :]
    dyl = jnp.asarray(dy)[None, :]
    stl = jnp.asarray(LSTART[ll], f32)[None, :]

    feats = [feat0, feat1, feat2, feat3]
    ft = jnp.concatenate(
        [feats[l][0].reshape(NC, ED, -1) for l in range(NL)], axis=2)
    ftpad = jnp.concatenate(
        [ft, jnp.zeros((NC, ED, NVP - NV), f32)], axis=2)

    l2i = _round_bf16(lidar2img[0])
    vp_b2 = vp_b[None, :]
    out_b2 = out_b[None, :]
    out_w_t = out_w.T

    offx, offy, aw = _run_a1(qpad, ppad, sowx, sobx, sowy, soby, aww, awb)
    idx, w, maskq = _run_a2(offx, offy, aw, rp_pad, l2i, img_size,
                            (wl, hl, dxl, dyl, stl))
    table = _run_b(ftpad, vp_w, vp_b2)
    table2 = table.reshape(NC * NVP * NH, HD)
    pre = _run_sc(idx.reshape(-1), w.reshape(-1), table2)
    pre4 = pre.reshape(NC, NH, NQP, HD)
    res = _run_d(pre4, maskq, out_w_t, out_b2)
    return res[:NQ][None]


# speedup vs baseline: 27.7547x; 1.0059x over previous
import jax
import jax.numpy as jnp
import numpy as np
from jax import lax
from jax.experimental import pallas as pl
from jax.experimental.pallas import tpu as pltpu
from jax.experimental.pallas import tpu_sc as plsc

B = 1
NQ = 2500
NQP = 2560
ED = 256
NH = 8
NL = 4
NP = 4
NC = 6
HD = ED // NH
SS = np.array([[64, 112], [32, 56], [16, 28], [8, 14]], dtype=np.int64)
LSTART = np.array([0, 7168, 8960, 9408], dtype=np.int64)
NV = 9520
NVP = 9600
NK = NL * NP * 4

ITEMS = NC * NH * NQP
NSUB = 32
IT_PER = ITEMS // NSUB
IB = 16
NBLK = IT_PER // IB

QB1 = 256
QB2 = 256
NVB = 640
QBD = 256

_f32 = jnp.float32
_i32 = jnp.int32


def _a1_body(qr, pr, wxr, bxr, wyr, byr, war, bar, oxr, oyr, oar):
    q = qr[...] + pr[...]
    dn = (((1,), (1,)), ((), ()))
    offx = lax.dot_general(q, wxr[0], dn, preferred_element_type=_f32) + bxr[0]
    offy = lax.dot_general(q, wyr[0], dn, preferred_element_type=_f32) + byr[0]
    a = lax.dot_general(q, war[0], dn, preferred_element_type=_f32) + bar[0]
    m = jnp.max(a, axis=1, keepdims=True)
    e = jnp.exp(a - m)
    s = jnp.sum(e, axis=1, keepdims=True)
    oxr[0] = offx
    oyr[0] = offy
    oar[0] = e * (4.0 / s)


def _run_a1(qpad, ppad, sowx, sobx, sowy, soby, aww, awb):
    grid = (NH, NQP // QB1)
    return pl.pallas_call(
        _a1_body,
        grid=grid,
        in_specs=[
            pl.BlockSpec((QB1, ED), lambda h, qi: (qi, 0)),
            pl.BlockSpec((QB1, ED), lambda h, qi: (qi, 0)),
            pl.BlockSpec((1, NK, ED), lambda h, qi: (h, 0, 0)),
            pl.BlockSpec((1, 1, NK), lambda h, qi: (h, 0, 0)),
            pl.BlockSpec((1, NK, ED), lambda h, qi: (h, 0, 0)),
            pl.BlockSpec((1, 1, NK), lambda h, qi: (h, 0, 0)),
            pl.BlockSpec((1, NK, ED), lambda h, qi: (h, 0, 0)),
            pl.BlockSpec((1, 1, NK), lambda h, qi: (h, 0, 0)),
        ],
        out_specs=[
            pl.BlockSpec((1, QB1, NK), lambda h, qi: (h, qi, 0)),
            pl.BlockSpec((1, QB1, NK), lambda h, qi: (h, qi, 0)),
            pl.BlockSpec((1, QB1, NK), lambda h, qi: (h, qi, 0)),
        ],
        out_shape=[jax.ShapeDtypeStruct((NH, NQP, NK), _f32)] * 3,
    )(qpad, ppad, sowx, sobx, sowy, soby, aww, awb)


def _a2_body(oxr, oyr, awr, rpr, l2ir, imgr, wlr, hlr, dxr, dyr, str_,
             idxr, wr, mr):
    c = pl.program_id(0)
    h = pl.program_id(2)
    rp = rpr[...]
    cam = []
    for i in range(3):
        acc = rp[:, 0:1] * l2ir[c, i, 0]
        for j in range(1, 4):
            acc = acc + rp[:, j:j + 1] * l2ir[c, i, j]
        cam.append(acc)
    depth = cam[2]
    denom = depth + 1e-5
    img_h = imgr[0].astype(_f32)
    img_w = imgr[1].astype(_f32)
    un = cam[0] / denom / img_w
    vn = cam[1] / denom / img_h
    maskf = ((depth > 1e-5) & (un > 0.0) & (un < 1.0)
             & (vn > 0.0) & (vn < 1.0)).astype(_f32)

    wl = wlr[...]
    hl = hlr[...]
    dxl = dxr[...]
    dyl = dyr[...]
    stl = str_[...]

    x = un * wl + oxr[0] - 0.5
    y = vn * hl + oyr[0] - 0.5
    x0 = jnp.floor(x)
    y0 = jnp.floor(y)
    fx = x - x0
    fy = y - y0
    xc = x0 + dxl
    yc = y0 + dyl
    wx = jnp.where(dxl > 0.5, fx, 1.0 - fx)
    wy = jnp.where(dyl > 0.5, fy, 1.0 - fy)
    inb = ((xc >= 0.0) & (xc <= wl - 1.0)
           & (yc >= 0.0) & (yc <= hl - 1.0)).astype(_f32)
    wtot = awr[0] * wx * wy * inb * maskf
    xci = jnp.clip(xc, 0.0, wl - 1.0)
    yci = jnp.clip(yc, 0.0, hl - 1.0)
    row_l = stl + yci * wl + xci
    gidx = row_l.astype(_i32) * NH + (c * NVP * NH + h)
    idxr[0, 0] = gidx
    wr[0, 0] = wtot
    mr[0] = maskf


def _run_a2(offx, offy, aw, rp_pad, l2i, img, lane_consts):
    wl, hl, dxl, dyl, stl = lane_consts
    grid = (NC, NQP // QB2, NH)
    lane_spec = pl.BlockSpec((1, NK), lambda c, qi, h: (0, 0))
    return pl.pallas_call(
        _a2_body,
        grid=grid,
        in_specs=[
            pl.BlockSpec((1, QB2, NK), lambda c, qi, h: (h, qi, 0)),
            pl.BlockSpec((1, QB2, NK), lambda c, qi, h: (h, qi, 0)),
            pl.BlockSpec((1, QB2, NK), lambda c, qi, h: (h, qi, 0)),
            pl.BlockSpec((QB2, 4), lambda c, qi, h: (qi, 0)),
            pl.BlockSpec(memory_space=pltpu.SMEM),
            pl.BlockSpec(memory_space=pltpu.SMEM),
            lane_spec, lane_spec, lane_spec, lane_spec, lane_spec,
        ],
        out_specs=[
            pl.BlockSpec((1, 1, QB2, NK), lambda c, qi, h: (c, h, qi, 0)),
            pl.BlockSpec((1, 1, QB2, NK), lambda c, qi, h: (c, h, qi, 0)),
            pl.BlockSpec((1, QB2, 1), lambda c, qi, h: (c, qi, 0)),
        ],
        out_shape=[
            jax.ShapeDtypeStruct((NC, NH, NQP, NK), _i32),
            jax.ShapeDtypeStruct((NC, NH, NQP, NK), _f32),
            jax.ShapeDtypeStruct((NC, NQP, 1), _f32),
        ],
    )(offx, offy, aw, rp_pad, l2i, img, wl, hl, dxl, dyl, stl)


def _b_body(fr, vwr, vbr, outr):
    dn = (((0,), (1,)), ((), ()))
    outr[0] = lax.dot_general(fr[0], vwr[...], dn,
                              preferred_element_type=_f32) + vbr[...]


def _run_b(ftpad, vp_w, vp_b2):
    grid = (NC, NVP // NVB)
    return pl.pallas_call(
        _b_body,
        grid=grid,
        in_specs=[
            pl.BlockSpec((1, ED, NVB), lambda c, vi: (c, 0, vi)),
            pl.BlockSpec((ED, ED), lambda c, vi: (0, 0)),
            pl.BlockSpec((1, ED), lambda c, vi: (0, 0)),
        ],
        out_specs=pl.BlockSpec((1, NVB, ED), lambda c, vi: (c, vi, 0)),
        out_shape=jax.ShapeDtypeStruct((NC, NVP, ED), _f32),
    )(ftpad, vp_w, vp_b2)


def _sc_body(idx_hbm, w_hbm, table_hbm, out_hbm,
             idxv0, wv0, rowsv0, idxv1, wv1, rowsv1, outv, sem0, sem1):
    cid = lax.axis_index("c")
    sid = lax.axis_index("s")
    wid = sid * 2 + cid
    base = wid * IT_PER
    bufs = ((idxv0, wv0, rowsv0, sem0), (idxv1, wv1, rowsv1, sem1))

    def fire(b, s):
        idxv, wv, rowsv, sem = bufs[s]
        ib = base + b * IB
        pltpu.sync_copy(idx_hbm.at[pl.ds(ib * NK, IB * NK)], idxv)
        pltpu.sync_copy(w_hbm.at[pl.ds(ib * NK, IB * NK)], wv)
        for g in range(IB * NK // 128):
            pltpu.async_copy(
                table_hbm.at[idxv.at[pl.ds(g * 128, 128)]],
                rowsv.at[pl.ds(g * 128, 128)], sem)

    def consume(b, s):
        idxv, wv, rowsv, sem = bufs[s]
        pltpu.make_async_copy(
            table_hbm.at[pl.ds(0, IB * NK)], rowsv, sem).wait()

        @pl.loop(0, IB)
        def _item(i):
            def kstep(k4, acc):
                a0, a1 = acc
                for j in range(4):
                    flat = i * NK + k4 * 4 + j
                    wk = plsc.load_gather(wv, [jnp.full((16,), flat, _i32)])
                    r0 = rowsv[flat, pl.ds(0, 16)]
                    r1 = rowsv[flat, pl.ds(16, 16)]
                    a0 = a0 + wk * r0
                    a1 = a1 + wk * r1
                return (a0, a1)

            a0, a1 = lax.fori_loop(
                0, NK // 4, kstep,
                (jnp.zeros((16,), _f32), jnp.zeros((16,), _f32)))
            outv[i, pl.ds(0, 16)] = a0
            outv[i, pl.ds(16, 16)] = a1

        pltpu.sync_copy(outv, out_hbm.at[pl.ds(base + b * IB, IB)])

    fire(0, 0)
    fire(1, 1)

    @pl.loop(0, NBLK // 2)
    def _pair(bb):
        for s in range(2):
            b = bb * 2 + s
            consume(b, s)

            @pl.when(b + 2 < NBLK)
            def _():
                fire(b + 2, s)


def _run_sc(idx_flat, w_flat, table):
    mesh = plsc.VectorSubcoreMesh(core_axis_name="c", subcore_axis_name="s")
    f = pl.kernel(
        _sc_body,
        out_type=jax.ShapeDtypeStruct((ITEMS, HD), _f32),
        mesh=mesh,
        scratch_types=[
            pltpu.VMEM((IB * NK,), _i32),
            pltpu.VMEM((IB * NK,), _f32),
            pltpu.VMEM((IB * NK, HD), _f32),
            pltpu.VMEM((IB * NK,), _i32),
            pltpu.VMEM((IB * NK,), _f32),
            pltpu.VMEM((IB * NK, HD), _f32),
            pltpu.VMEM((IB, HD), _f32),
            pltpu.SemaphoreType.DMA,
            pltpu.SemaphoreType.DMA,
        ],
        compiler_params=pltpu.CompilerParams(
            needs_layout_passes=False, use_tc_tiling_on_sc=False),
    )
    return f(idx_flat, w_flat, table)


def _d_body(prer, mr, owr, obr, outr):
    h = pl.program_id(1)
    fused = jnp.sum(prer[...], axis=0)[0]
    m = mr[...]
    valid = jnp.sum(m, axis=0)
    scale = 1.0 / jnp.maximum(valid, 1.0)
    ind = (valid > 0.0).astype(_f32)
    part = jnp.dot(fused * scale, owr[...], preferred_element_type=_f32)

    @pl.when(h == 0)
    def _():
        outr[...] = part + ind * obr[...]

    @pl.when(h != 0)
    def _():
        outr[...] = outr[...] + part


def _run_d(pre, maskq, out_w_t, out_b2):
    grid = (NQP // QBD, NH)
    return pl.pallas_call(
        _d_body,
        grid=grid,
        in_specs=[
            pl.BlockSpec((NC, 1, QBD, HD), lambda qi, h: (0, h, qi, 0)),
            pl.BlockSpec((NC, QBD, 1), lambda qi, h: (0, qi, 0)),
            pl.BlockSpec((HD, ED), lambda qi, h: (h, 0)),
            pl.BlockSpec((1, ED), lambda qi, h: (0, 0)),
        ],
        out_specs=pl.BlockSpec((QBD, ED), lambda qi, h: (qi, 0)),
        out_shape=jax.ShapeDtypeStruct((NQP, ED), _f32),
    )(pre, maskq, out_w_t, out_b2)


def kernel(query, query_pos, feat0, feat1, feat2, feat3, lidar2img,
           reference_points, img_size, so_w, so_b, aw_w, aw_b, vp_w, vp_b,
           out_w, out_b):
    f32 = _f32
    qpad = jnp.concatenate(
        [query[0], jnp.zeros((NQP - NQ, ED), f32)], axis=0)
    ppad = jnp.concatenate(
        [query_pos[0], jnp.zeros((NQP - NQ, ED), f32)], axis=0)
    def _round_bf16(x):
        u = jax.lax.bitcast_convert_type(x, jnp.uint32)
        u = (u + jnp.uint32(0x7FFF) + ((u >> 16) & jnp.uint32(1)))
        u = u & jnp.uint32(0xFFFF0000)
        return jax.lax.bitcast_convert_type(u, f32)

    rp1 = jnp.concatenate(
        [reference_points, jnp.ones((NQ, 1), f32)], axis=1)
    rp1 = _round_bf16(rp1)
    rp_pad = jnp.concatenate([rp1, jnp.zeros((NQP - NQ, 4), f32)], axis=0)

    kk = np.arange(NK)
    ll = kk // 16
    pp = (kk // 4) % 4
    cc = kk % 4
    dx = (cc % 2).astype(np.float32)
    dy = (cc // 2).astype(np.float32)
    hh = np.arange(NH)[:, None]
    rx = (((hh * NL + ll[None]) * NP + pp[None]) * 2 + 0).astype(np.int32)
    ra = ((hh * NL + ll[None]) * NP + pp[None]).astype(np.int32)
    sowx = so_w[rx]
    sowy = so_w[rx + 1]
    sobx = so_b[rx][:, None, :]
    soby = so_b[rx + 1][:, None, :]
    aww = aw_w[ra]
    awb = aw_b[ra][:, None, :]

    wl = jnp.asarray(SS[ll, 1], f32)[None, :]
    hl = jnp.asarray(SS[ll, 0], f32)[None, :]
    dxl = jnp.asarray(dx)[None, :]
    dyl = jnp.asarray(dy)[None, :]
    stl = jnp.asarray(LSTART[ll], f32)[None, :]

    feats = [feat0, feat1, feat2, feat3]
    ft = jnp.concatenate(
        [feats[l][0].reshape(NC, ED, -1) for l in range(NL)], axis=2)
    ftpad = jnp.concatenate(
        [ft, jnp.zeros((NC, ED, NVP - NV), f32)], axis=2)

    l2i = _round_bf16(lidar2img[0])
    vp_b2 = vp_b[None, :]
    out_b2 = out_b[None, :]
    out_w_t = out_w.T

    offx, offy, aw = _run_a1(qpad, ppad, sowx, sobx, sowy, soby, aww, awb)
    idx, w, maskq = _run_a2(offx, offy, aw, rp_pad, l2i, img_size,
                            (wl, hl, dxl, dyl, stl))
    table = _run_b(ftpad, vp_w, vp_b2)
    table2 = table.reshape(NC * NVP * NH, HD)
    pre = _run_sc(idx.reshape(-1), w.reshape(-1), table2)
    pre4 = pre.reshape(NC, NH, NQP, HD)
    res = _run_d(pre4, maskq, out_w_t, out_b2)
    return res[:NQ][None]
